# Initial kernel scaffold; baseline (speedup 1.0000x reference)
import jax
import jax.numpy as jnp
from jax.experimental import pallas as pl


def kernel(x, edge_index, edge_type, goalVec, goalObjectsVec, Wm0, bm0, Ws0, bs0, Wg0, bg0, Wm1, bm1, Ws1, bs1, Wg1, bg1, Wm2, bm2, Ws2, bs2, Wg2, bg2, W_att, b_att, W_e, b_e, W1, b1, W2, b2, W3, b3):
    raise NotImplementedError("write your pallas kernel here")


# speedup vs baseline: 4.5203x; 4.5203x over previous
import functools

import jax
import jax.numpy as jnp
from jax import lax
from jax.experimental import pallas as pl
from jax.experimental.pallas import tpu as pltpu
from jax.experimental.pallas import tpu_sc as plsc

_N = 10000
_E = 320000
_R = 4
_H = 256
_HALF = 128
_CHUNK = 128
_NCHUNKS = 2528
_EPAD = _NCHUNKS * _CHUNK
_TILES = 16
_ACC_ROWS = 10240
_DUMMY = _N
_CNT_ROWS = _R * _ACC_ROWS
_BN = 2000


def _cnt_body(tab_hbm, et_hbm, dst_hbm, out_hbm, acc_sp, rows_v, gidx_v,
              didx_v, zb_v, sem):
    c = lax.axis_index("c")
    s = lax.axis_index("s")

    def fillz(i, _):
        zb_v[i // 8, pl.ds((i % 8) * 16, 16)] = jnp.zeros((16,), jnp.float32)
        return 0
    lax.fori_loop(0, 1024, fillz, 0)

    def zchunk(i, _):
        pltpu.sync_copy(zb_v, acc_sp.at[pl.ds((s * 5 + i) * 128, 128), :])
        return 0
    lax.fori_loop(0, 5, zchunk, 0)
    plsc.subcore_barrier()

    w = s * 2 + c

    def body(j, _):
        off = (w * 79 + j) * _CHUNK
        pltpu.sync_copy(et_hbm.at[pl.ds(off, _CHUNK)], gidx_v)
        pltpu.sync_copy(dst_hbm.at[pl.ds(off, _CHUNK)], didx_v)
        pltpu.async_copy(tab_hbm.at[gidx_v], rows_v, sem).wait()
        pltpu.sync_copy(rows_v, acc_sp.at[didx_v], add=True)
        return 0
    lax.fori_loop(0, 79, body, 0)
    plsc.subcore_barrier()

    def wchunk(i, _):
        r0 = (s * 5 + i) * 128
        pltpu.sync_copy(acc_sp.at[pl.ds(r0, 128), :], rows_v)

        @pl.when(c == 0)
        def _():
            pltpu.sync_copy(rows_v, out_hbm.at[0, pl.ds(r0, 128), :])

        @pl.when(c == 1)
        def _():
            pltpu.sync_copy(rows_v, out_hbm.at[1, pl.ds(r0, 128), :])
        return 0
    lax.fori_loop(0, 5, wchunk, 0)


def _edge_counts(onehot_tab, et_pad, dst_pad):
    k = pl.kernel(
        _cnt_body,
        out_type=jax.ShapeDtypeStruct((2, _ACC_ROWS, _HALF), jnp.float32),
        mesh=plsc.VectorSubcoreMesh(core_axis_name="c", subcore_axis_name="s"),
        scratch_types=[
            pltpu.VMEM_SHARED((_ACC_ROWS, _HALF), jnp.float32),
            pltpu.VMEM((_CHUNK, _HALF), jnp.float32),
            pltpu.VMEM((_CHUNK,), jnp.int32),
            pltpu.VMEM((_CHUNK,), jnp.int32),
            pltpu.VMEM((128, 128), jnp.float32),
            pltpu.SemaphoreType.DMA,
        ],
    )
    return k(onehot_tab, et_pad, dst_pad)


def _agg_body(xs_lo_hbm, xs_hi_hbm, gidx_hbm, dst_hbm, out_lo_hbm, out_hi_hbm,
              acc_sp, rows_v, gidx_v, didx_v, zb_v, sem):
    c = lax.axis_index("c")
    s = lax.axis_index("s")

    def fillz(i, _):
        zb_v[i // 8, pl.ds((i % 8) * 16, 16)] = jnp.zeros((16,), jnp.float32)
        return 0
    lax.fori_loop(0, 1024, fillz, 0)

    def zchunk(i, _):
        pltpu.sync_copy(zb_v, acc_sp.at[pl.ds((s * 5 + i) * 128, 128), :])
        return 0
    lax.fori_loop(0, 5, zchunk, 0)
    plsc.subcore_barrier()

    def edge_loop(table_hbm):
        def body(j, _):
            off = (s * 158 + j) * _CHUNK
            pltpu.sync_copy(gidx_hbm.at[pl.ds(off, _CHUNK)], gidx_v)
            pltpu.sync_copy(dst_hbm.at[pl.ds(off, _CHUNK)], didx_v)
            pltpu.async_copy(table_hbm.at[gidx_v], rows_v, sem).wait()
            pltpu.sync_copy(rows_v, acc_sp.at[didx_v], add=True)
            return 0
        lax.fori_loop(0, 158, body, 0)

    @pl.when(c == 0)
    def _():
        edge_loop(xs_lo_hbm)

    @pl.when(c == 1)
    def _():
        edge_loop(xs_hi_hbm)

    plsc.subcore_barrier()

    def wchunk(i, _):
        r0 = (s * 5 + i) * 128
        pltpu.sync_copy(acc_sp.at[pl.ds(r0, 128), :], rows_v)

        @pl.when(c == 0)
        def _():
            pltpu.sync_copy(rows_v, out_lo_hbm.at[pl.ds(r0, 128), :])

        @pl.when(c == 1)
        def _():
            pltpu.sync_copy(rows_v, out_hi_hbm.at[pl.ds(r0, 128), :])
        return 0
    lax.fori_loop(0, 5, wchunk, 0)


def _edge_aggregate(xs_lo, xs_hi, gidx_pad, dst_pad):
    k = pl.kernel(
        _agg_body,
        out_type=[jax.ShapeDtypeStruct((_ACC_ROWS, _HALF), jnp.float32),
                  jax.ShapeDtypeStruct((_ACC_ROWS, _HALF), jnp.float32)],
        mesh=plsc.VectorSubcoreMesh(core_axis_name="c", subcore_axis_name="s"),
        scratch_types=[
            pltpu.VMEM_SHARED((_ACC_ROWS, _HALF), jnp.float32),
            pltpu.VMEM((_CHUNK, _HALF), jnp.float32),
            pltpu.VMEM((_CHUNK,), jnp.int32),
            pltpu.VMEM((_CHUNK,), jnp.int32),
            pltpu.VMEM((128, 128), jnp.float32),
            pltpu.SemaphoreType.DMA,
        ],
    )
    return k(xs_lo, xs_hi, gidx_pad, dst_pad)


def _transform(h, Wm, Ws, bs_row):
    din = h.shape[1]

    def body(h_ref, wm_ref, ws_ref, bs_ref, xlo_ref, xhi_ref, selfh_ref):
        hb = h_ref[...]
        for r in range(_R):
            t = jnp.dot(hb, wm_ref[r], preferred_element_type=jnp.float32)
            xlo_ref[r] = t[:, :_HALF]
            xhi_ref[r] = t[:, _HALF:]
        selfh_ref[...] = (
            jnp.dot(hb, ws_ref[...], preferred_element_type=jnp.float32)
            + bs_ref[...])

    return pl.pallas_call(
        body,
        grid=(_N // _BN,),
        in_specs=[
            pl.BlockSpec((_BN, din), lambda g: (g, 0)),
            pl.BlockSpec((_R, din, _H), lambda g: (0, 0, 0)),
            pl.BlockSpec((din, _H), lambda g: (0, 0)),
            pl.BlockSpec((1, _H), lambda g: (0, 0)),
        ],
        out_specs=[
            pl.BlockSpec((_R, _BN, _HALF), lambda g: (0, g, 0)),
            pl.BlockSpec((_R, _BN, _HALF), lambda g: (0, g, 0)),
            pl.BlockSpec((_BN, _H), lambda g: (g, 0)),
        ],
        out_shape=[
            jax.ShapeDtypeStruct((_R, _N, _HALF), jnp.float32),
            jax.ShapeDtypeStruct((_R, _N, _HALF), jnp.float32),
            jax.ShapeDtypeStruct((_N, _H), jnp.float32),
        ],
    )(h, Wm, Ws, bs_row)


def _combine(S_lo, S_hi, selfh, cnt4, bm, Wg, bg_row):
    def body(slo_ref, shi_ref, selfh_ref, cnt_ref, bm_ref, wg_ref, bg_ref,
             out_ref):
        cnt = cnt_ref[0, :, :_R] + cnt_ref[1, :, :_R]
        deg = jnp.sum(cnt, axis=1)
        S = jnp.concatenate([slo_ref[...], shi_ref[...]], axis=1)
        aggpre = S + jnp.dot(cnt, bm_ref[...],
                             preferred_element_type=jnp.float32)
        agg = aggpre / jnp.maximum(deg, 1.0)[:, None]
        sh = selfh_ref[...]
        zin = jnp.concatenate([sh, agg], axis=1)
        z = jax.nn.sigmoid(
            jnp.dot(zin, wg_ref[...], preferred_element_type=jnp.float32)
            + bg_ref[...])
        out_ref[...] = jnp.maximum(z * agg + (1.0 - z) * sh, 0.0)

    return pl.pallas_call(
        body,
        grid=(_N // _BN,),
        in_specs=[
            pl.BlockSpec((_BN, _HALF), lambda g: (g, 0)),
            pl.BlockSpec((_BN, _HALF), lambda g: (g, 0)),
            pl.BlockSpec((_BN, _H), lambda g: (g, 0)),
            pl.BlockSpec((2, _BN, _HALF), lambda g: (0, g, 0)),
            pl.BlockSpec((_R, _H), lambda g: (0, 0)),
            pl.BlockSpec((2 * _H, _H), lambda g: (0, 0)),
            pl.BlockSpec((1, _H), lambda g: (0, 0)),
        ],
        out_specs=pl.BlockSpec((_BN, _H), lambda g: (g, 0)),
        out_shape=jax.ShapeDtypeStruct((_N, _H), jnp.float32),
    )(S_lo, S_hi, selfh, cnt4, bm, Wg, bg_row)


def _head(h, gv_row, gov_row, W_e, be_row, wa_row, W1, b1_row, W2, b2_row,
          W3, b3_row):
    def body(h_ref, gv_ref, gov_ref, we_ref, be_ref, wa_ref, w1_ref, b1_ref,
             w2_ref, b2_ref, w3_ref, b3_ref, out_ref):
        h = h_ref[...]
        gobj_part = wa_ref[...]
        gv_e = jnp.tanh(
            jnp.dot(gv_ref[...], we_ref[...],
                    preferred_element_type=jnp.float32) + be_ref[...])
        s = jnp.sum(h * gobj_part[:, :_H], axis=1, keepdims=True)
        m = jnp.max(s)
        e = jnp.exp(s - m)
        scene = jnp.sum(e * h, axis=0, keepdims=True) / jnp.sum(e)
        f = jnp.concatenate([scene, gv_e], axis=1)
        h1 = jnp.tanh(
            jnp.dot(f, w1_ref[...], preferred_element_type=jnp.float32)
            + b1_ref[...])
        h2 = jnp.tanh(
            jnp.dot(h1, w2_ref[...], preferred_element_type=jnp.float32)
            + b2_ref[...])
        out_ref[...] = jax.nn.sigmoid(
            jnp.dot(h2, w3_ref[...], preferred_element_type=jnp.float32)
            + b3_ref[...])

    return pl.pallas_call(
        body,
        out_shape=jax.ShapeDtypeStruct((1, 64), jnp.float32),
    )(h, gv_row, gov_row, W_e, be_row, wa_row, W1, b1_row, W2, b2_row,
      W3, b3_row)


def kernel(x, edge_index, edge_type, goalVec, goalObjectsVec,
           Wm0, bm0, Ws0, bs0, Wg0, bg0,
           Wm1, bm1, Ws1, bs1, Wg1, bg1,
           Wm2, bm2, Ws2, bs2, Wg2, bg2,
           W_att, b_att, W_e, b_e,
           W1, b1, W2, b2, W3, b3):
    src, dst = edge_index[0], edge_index[1]
    npad = _EPAD - _E
    gidx_pad = jnp.concatenate(
        [edge_type * _N + src, jnp.zeros((npad,), jnp.int32)])
    dst_pad = jnp.concatenate(
        [dst, jnp.full((npad,), _DUMMY, jnp.int32)])
    et_pad = jnp.concatenate(
        [edge_type, jnp.full((npad,), _R, jnp.int32)])
    onehot_tab = jnp.zeros((8, _HALF), jnp.float32).at[
        jnp.arange(_R), jnp.arange(_R)].set(1.0)

    cnt4 = _edge_counts(onehot_tab, et_pad, dst_pad)

    h = x
    layers = [(Wm0, bm0, Ws0, bs0, Wg0, bg0),
              (Wm1, bm1, Ws1, bs1, Wg1, bg1),
              (Wm2, bm2, Ws2, bs2, Wg2, bg2)]
    for (Wm, bm, Ws, bs, Wg, bg) in layers:
        xlo, xhi, selfh = _transform(h, Wm, Ws, bs.reshape(1, _H))
        S_lo, S_hi = _edge_aggregate(
            xlo.reshape(_R * _N, _HALF), xhi.reshape(_R * _N, _HALF),
            gidx_pad, dst_pad)
        h = _combine(S_lo, S_hi, selfh, cnt4, bm, Wg, bg.reshape(1, _H))

    out = _head(h, goalVec.reshape(1, -1), goalObjectsVec.reshape(1, -1),
                W_e, b_e.reshape(1, _H), W_att.reshape(1, 2 * _H),
                W1, b1.reshape(1, _H), W2, b2.reshape(1, _H),
                W3, b3.reshape(1, 64))
    return out.reshape(-1)


# speedup vs baseline: 9.6468x; 2.1341x over previous
import functools

import jax
import jax.numpy as jnp
from jax import lax
from jax.experimental import pallas as pl
from jax.experimental.pallas import tpu as pltpu
from jax.experimental.pallas import tpu_sc as plsc

_N = 10000
_E = 320000
_R = 4
_H = 256
_HALF = 128
_CHUNK = 128
_NCHUNKS = 2560
_EPAD = _NCHUNKS * _CHUNK
_TILES = 16
_TCH = _NCHUNKS // _TILES
_WCH = _NCHUNKS // (2 * _TILES)
_ACC_ROWS = 10240
_DUMMY = _N
_CREP = 1024
_GRP = 16
_BN = 2000


def _fill_zeros(buf2d):
    def fillz(i, _):
        buf2d[i // 8, pl.ds((i % 8) * 16, 16)] = jnp.zeros((16,), jnp.float32)
        return 0
    lax.fori_loop(0, 1024, fillz, 0)


def _zero_acc(acc_sp, zsrc, s):
    def zchunk(i, _):
        pltpu.sync_copy(zsrc, acc_sp.at[pl.ds((s * 5 + i) * 128, 128), :])
        return 0
    lax.fori_loop(0, 5, zchunk, 0)


def _acc_writeout(acc_sp, stage, out_hbm, s):
    def wchunk(i, _):
        r0 = (s * 5 + i) * 128
        pltpu.sync_copy(acc_sp.at[pl.ds(r0, 128), :], stage)
        pltpu.sync_copy(stage, out_hbm.at[pl.ds(r0, 128), :])
        return 0
    lax.fori_loop(0, 5, wchunk, 0)


def _edge_stream_pipeline(table_hbm, gidx2d_hbm, didx2d_hbm, acc_sp,
                          gidx_v, didx_v, rows_v, sem0, sem1, base, n):
    sems = (sem0, sem1)

    def fire(j, slot):
        pltpu.async_copy(table_hbm.at[gidx_v.at[j]], rows_v.at[slot],
                         sems[slot])

    def drain(slot):
        pltpu.make_async_copy(table_hbm.at[pl.ds(0, _CHUNK), :],
                              rows_v.at[slot], sems[slot]).wait()

    def scatter(j, slot):
        pltpu.sync_copy(rows_v.at[slot], acc_sp.at[didx_v.at[j]], add=True)

    def group(g, _):
        gb = base + g * _GRP
        pltpu.sync_copy(gidx2d_hbm.at[pl.ds(gb, _GRP), :], gidx_v)
        pltpu.sync_copy(didx2d_hbm.at[pl.ds(gb, _GRP), :], didx_v)
        fire(0, 0)

        def body(p, _):
            j0 = 2 * p
            fire(j0 + 1, 1)
            drain(0)
            scatter(j0, 0)
            fire(j0 + 2, 0)
            drain(1)
            scatter(j0 + 1, 1)
            return 0
        lax.fori_loop(0, _GRP // 2 - 1, body, 0)
        fire(_GRP - 1, 1)
        drain(0)
        scatter(_GRP - 2, 0)
        drain(1)
        scatter(_GRP - 1, 1)
        return 0
    lax.fori_loop(0, n // _GRP, group, 0)


def _cnt_body(tab_hbm, gidx2d_hbm, didx2d_hbm, out_hbm, acc_sp, gidx_v,
              didx_v, rows_v, sem0, sem1):
    c = lax.axis_index("c")
    s = lax.axis_index("s")

    _fill_zeros(rows_v.at[0])
    _zero_acc(acc_sp, rows_v.at[0], s)
    plsc.subcore_barrier()

    w = s * 2 + c
    _edge_stream_pipeline(tab_hbm, gidx2d_hbm, didx2d_hbm, acc_sp,
                          gidx_v, didx_v, rows_v, sem0, sem1,
                          w * _WCH, _WCH)
    plsc.subcore_barrier()

    @pl.when(c == 0)
    def _():
        _acc_writeout(acc_sp, rows_v.at[0], out_hbm.at[0], s)

    @pl.when(c == 1)
    def _():
        _acc_writeout(acc_sp, rows_v.at[0], out_hbm.at[1], s)


def _edge_counts(onehot_tab, cgid2d, dst2d):
    k = pl.kernel(
        _cnt_body,
        out_type=jax.ShapeDtypeStruct((2, _ACC_ROWS, _HALF), jnp.float32),
        mesh=plsc.VectorSubcoreMesh(core_axis_name="c", subcore_axis_name="s"),
        scratch_types=[
            pltpu.VMEM_SHARED((_ACC_ROWS, _HALF), jnp.float32),
            pltpu.VMEM((_GRP, _CHUNK), jnp.int32),
            pltpu.VMEM((_GRP, _CHUNK), jnp.int32),
            pltpu.VMEM((2, _CHUNK, _HALF), jnp.float32),
            pltpu.SemaphoreType.DMA,
            pltpu.SemaphoreType.DMA,
        ],
    )
    return k(onehot_tab, cgid2d, dst2d)


def _agg_body(xs_lo_hbm, xs_hi_hbm, gidx2d_hbm, didx2d_hbm, out_lo_hbm,
              out_hi_hbm, acc_sp, gidx_v, didx_v, rows_v, sem0, sem1):
    c = lax.axis_index("c")
    s = lax.axis_index("s")

    _fill_zeros(rows_v.at[0])
    _zero_acc(acc_sp, rows_v.at[0], s)
    plsc.subcore_barrier()

    @pl.when(c == 0)
    def _():
        _edge_stream_pipeline(xs_lo_hbm, gidx2d_hbm, didx2d_hbm, acc_sp,
                              gidx_v, didx_v, rows_v, sem0, sem1,
                              s * _TCH, _TCH)

    @pl.when(c == 1)
    def _():
        _edge_stream_pipeline(xs_hi_hbm, gidx2d_hbm, didx2d_hbm, acc_sp,
                              gidx_v, didx_v, rows_v, sem0, sem1,
                              s * _TCH, _TCH)

    plsc.subcore_barrier()

    @pl.when(c == 0)
    def _():
        _acc_writeout(acc_sp, rows_v.at[0], out_lo_hbm, s)

    @pl.when(c == 1)
    def _():
        _acc_writeout(acc_sp, rows_v.at[0], out_hi_hbm, s)


def _edge_aggregate(xs_lo, xs_hi, gidx2d, dst2d):
    k = pl.kernel(
        _agg_body,
        out_type=[jax.ShapeDtypeStruct((_ACC_ROWS, _HALF), jnp.float32),
                  jax.ShapeDtypeStruct((_ACC_ROWS, _HALF), jnp.float32)],
        mesh=plsc.VectorSubcoreMesh(core_axis_name="c", subcore_axis_name="s"),
        scratch_types=[
            pltpu.VMEM_SHARED((_ACC_ROWS, _HALF), jnp.float32),
            pltpu.VMEM((_GRP, _CHUNK), jnp.int32),
            pltpu.VMEM((_GRP, _CHUNK), jnp.int32),
            pltpu.VMEM((2, _CHUNK, _HALF), jnp.float32),
            pltpu.SemaphoreType.DMA,
            pltpu.SemaphoreType.DMA,
        ],
    )
    return k(xs_lo, xs_hi, gidx2d, dst2d)


def _transform(h, Wm, Ws, bs_row):
    din = h.shape[1]

    def body(h_ref, wm_ref, ws_ref, bs_ref, xlo_ref, xhi_ref, selfh_ref):
        hb = h_ref[...]
        for r in range(_R):
            t = jnp.dot(hb, wm_ref[r], preferred_element_type=jnp.float32)
            xlo_ref[r] = t[:, :_HALF]
            xhi_ref[r] = t[:, _HALF:]
        selfh_ref[...] = (
            jnp.dot(hb, ws_ref[...], preferred_element_type=jnp.float32)
            + bs_ref[...])

    return pl.pallas_call(
        body,
        grid=(_N // _BN,),
        in_specs=[
            pl.BlockSpec((_BN, din), lambda g: (g, 0)),
            pl.BlockSpec((_R, din, _H), lambda g: (0, 0, 0)),
            pl.BlockSpec((din, _H), lambda g: (0, 0)),
            pl.BlockSpec((1, _H), lambda g: (0, 0)),
        ],
        out_specs=[
            pl.BlockSpec((_R, _BN, _HALF), lambda g: (0, g, 0)),
            pl.BlockSpec((_R, _BN, _HALF), lambda g: (0, g, 0)),
            pl.BlockSpec((_BN, _H), lambda g: (g, 0)),
        ],
        out_shape=[
            jax.ShapeDtypeStruct((_R, _N, _HALF), jnp.float32),
            jax.ShapeDtypeStruct((_R, _N, _HALF), jnp.float32),
            jax.ShapeDtypeStruct((_N, _H), jnp.float32),
        ],
    )(h, Wm, Ws, bs_row)


def _combine(S_lo, S_hi, selfh, cnt4, bm, Wg, bg_row):
    def body(slo_ref, shi_ref, selfh_ref, cnt_ref, bm_ref, wg_ref, bg_ref,
             out_ref):
        cnt = cnt_ref[0, :, :_R] + cnt_ref[1, :, :_R]
        deg = jnp.sum(cnt, axis=1)
        S = jnp.concatenate([slo_ref[...], shi_ref[...]], axis=1)
        aggpre = S + jnp.dot(cnt, bm_ref[...],
                             preferred_element_type=jnp.float32)
        agg = aggpre / jnp.maximum(deg, 1.0)[:, None]
        sh = selfh_ref[...]
        zin = jnp.concatenate([sh, agg], axis=1)
        z = jax.nn.sigmoid(
            jnp.dot(zin, wg_ref[...], preferred_element_type=jnp.float32)
            + bg_ref[...])
        out_ref[...] = jnp.maximum(z * agg + (1.0 - z) * sh, 0.0)

    return pl.pallas_call(
        body,
        grid=(_N // _BN,),
        in_specs=[
            pl.BlockSpec((_BN, _HALF), lambda g: (g, 0)),
            pl.BlockSpec((_BN, _HALF), lambda g: (g, 0)),
            pl.BlockSpec((_BN, _H), lambda g: (g, 0)),
            pl.BlockSpec((2, _BN, _HALF), lambda g: (0, g, 0)),
            pl.BlockSpec((_R, _H), lambda g: (0, 0)),
            pl.BlockSpec((2 * _H, _H), lambda g: (0, 0)),
            pl.BlockSpec((1, _H), lambda g: (0, 0)),
        ],
        out_specs=pl.BlockSpec((_BN, _H), lambda g: (g, 0)),
        out_shape=jax.ShapeDtypeStruct((_N, _H), jnp.float32),
    )(S_lo, S_hi, selfh, cnt4, bm, Wg, bg_row)


def _head(h, gv_row, gov_row, W_e, be_row, wa_row, W1, b1_row, W2, b2_row,
          W3, b3_row):
    def body(h_ref, gv_ref, gov_ref, we_ref, be_ref, wa_ref, w1_ref, b1_ref,
             w2_ref, b2_ref, w3_ref, b3_ref, out_ref):
        h = h_ref[...]
        gobj_part = wa_ref[...]
        gv_e = jnp.tanh(
            jnp.dot(gv_ref[...], we_ref[...],
                    preferred_element_type=jnp.float32) + be_ref[...])
        s = jnp.sum(h * gobj_part[:, :_H], axis=1, keepdims=True)
        m = jnp.max(s)
        e = jnp.exp(s - m)
        scene = jnp.sum(e * h, axis=0, keepdims=True) / jnp.sum(e)
        f = jnp.concatenate([scene, gv_e], axis=1)
        h1 = jnp.tanh(
            jnp.dot(f, w1_ref[...], preferred_element_type=jnp.float32)
            + b1_ref[...])
        h2 = jnp.tanh(
            jnp.dot(h1, w2_ref[...], preferred_element_type=jnp.float32)
            + b2_ref[...])
        out_ref[...] = jax.nn.sigmoid(
            jnp.dot(h2, w3_ref[...], preferred_element_type=jnp.float32)
            + b3_ref[...])

    return pl.pallas_call(
        body,
        out_shape=jax.ShapeDtypeStruct((1, 64), jnp.float32),
    )(h, gv_row, gov_row, W_e, be_row, wa_row, W1, b1_row, W2, b2_row,
      W3, b3_row)


def kernel(x, edge_index, edge_type, goalVec, goalObjectsVec,
           Wm0, bm0, Ws0, bs0, Wg0, bg0,
           Wm1, bm1, Ws1, bs1, Wg1, bg1,
           Wm2, bm2, Ws2, bs2, Wg2, bg2,
           W_att, b_att, W_e, b_e,
           W1, b1, W2, b2, W3, b3):
    src, dst = edge_index[0], edge_index[1]
    npad = _EPAD - _E
    gidx2d = jnp.concatenate(
        [edge_type * _N + src, jnp.zeros((npad,), jnp.int32)]
    ).reshape(_NCHUNKS, _CHUNK)
    dst2d = jnp.concatenate(
        [dst, jnp.full((npad,), _DUMMY, jnp.int32)]
    ).reshape(_NCHUNKS, _CHUNK)
    cgid2d = jnp.concatenate(
        [edge_type * _CREP + jnp.arange(_E, dtype=jnp.int32) % _CREP,
         jnp.zeros((npad,), jnp.int32)]
    ).reshape(_NCHUNKS, _CHUNK)
    onehot_tab = jnp.repeat(
        jnp.zeros((_R, _HALF), jnp.float32).at[
            jnp.arange(_R), jnp.arange(_R)].set(1.0),
        _CREP, axis=0)

    cnt4 = _edge_counts(onehot_tab, cgid2d, dst2d)

    h = x
    layers = [(Wm0, bm0, Ws0, bs0, Wg0, bg0),
              (Wm1, bm1, Ws1, bs1, Wg1, bg1),
              (Wm2, bm2, Ws2, bs2, Wg2, bg2)]
    for (Wm, bm, Ws, bs, Wg, bg) in layers:
        xlo, xhi, selfh = _transform(h, Wm, Ws, bs.reshape(1, _H))
        S_lo, S_hi = _edge_aggregate(
            xlo.reshape(_R * _N, _HALF), xhi.reshape(_R * _N, _HALF),
            gidx2d, dst2d)
        h = _combine(S_lo, S_hi, selfh, cnt4, bm, Wg, bg.reshape(1, _H))

    out = _head(h, goalVec.reshape(1, -1), goalObjectsVec.reshape(1, -1),
                W_e, b_e.reshape(1, _H), W_att.reshape(1, 2 * _H),
                W1, b1.reshape(1, _H), W2, b2.reshape(1, _H),
                W3, b3.reshape(1, 64))
    return out.reshape(-1)
